# baseline (speedup 1.0000x reference)
import functools

import jax
import jax.numpy as jnp
from jax import lax
from jax.experimental import pallas as pl
from jax.experimental.pallas import tpu as pltpu
from jax.experimental.pallas import tpu_sc as plsc

N = 50000
VOCAB = 100000
D = 256
H = 512
C = 32
G = 512

NC = 2
NS = 16
NW = NC * NS
BLK = 128
NBLK = -(-N // (NW * BLK))
CHUNK = NBLK * BLK
NP = NW * CHUNK
GP = G + 8
LANES = 16
NCH = D // LANES
ZR = 64


def _sc_pool_body(x_hbm, b_hbm, table_hbm, out_hbm,
                  x_v, b_v, xidx0, xidx1, rows0, rows1, zbuf, fbuf,
                  sem0, sem1, zsem):
    c = lax.axis_index("c")
    s = lax.axis_index("s")
    wid = s * NC + c

    pltpu.sync_copy(x_hbm.at[wid], x_v)
    pltpu.sync_copy(b_hbm.at[wid], b_v)

    xidx = (xidx0, xidx1)
    rows = (rows0, rows1)
    sems = (sem0, sem1)

    def copy_idx(row, dst_ref):
        for j in range(BLK // LANES):
            dst_ref[pl.ds(j * LANES, LANES)] = x_v[row, pl.ds(j * LANES, LANES)]

    def start_gather(blk, p):
        copy_idx(blk, xidx[p])
        pltpu.async_copy(table_hbm.at[xidx[p]], rows[p], sems[p])

    def wait_gather(p):
        pltpu.make_async_copy(table_hbm.at[xidx[p]], rows[p], sems[p]).wait()

    zv = jnp.zeros((LANES,), jnp.float32)

    def zero_zbuf_row(r, _):
        for j in range(NCH):
            zbuf[r, pl.ds(j * LANES, LANES)] = zv
        return 0

    lax.fori_loop(0, ZR, zero_zbuf_row, 0)
    start_gather(0, 0)
    for k in range(G // ZR):
        pltpu.async_copy(zbuf, out_hbm.at[c, s, pl.ds(k * ZR, ZR)], zsem)
    for k in range(G // ZR):
        pltpu.make_async_copy(zbuf, out_hbm.at[c, s, pl.ds(k * ZR, ZR)],
                              zsem).wait()

    def flush(seg, av):
        for j in range(NCH):
            fbuf[pl.ds(j * LANES, LANES)] = av[j]
        pltpu.sync_copy(fbuf, out_hbm.at[c, s, seg])

    def accumulate(rows_ref, blk, carry):
        def group(g, carry):
            seg_cur, av = carry
            segs = b_v[blk, pl.ds(g * LANES, LANES)]
            for l in range(LANES):
                seg_l = segs[l]
                is_new = seg_l != seg_cur

                @pl.when(is_new)
                def _(seg_cur=seg_cur, av=av):
                    flush(seg_cur, av)

                keep = 1.0 - is_new.astype(jnp.float32)
                r = g * LANES + l
                av = tuple(av[j] * keep + rows_ref[r, pl.ds(j * LANES, LANES)]
                           for j in range(NCH))
                seg_cur = seg_l
            return (seg_cur, av)

        return lax.fori_loop(0, BLK // LANES, group, carry)

    carry0 = (b_v[0, pl.ds(0, LANES)][0],
              tuple(jnp.zeros((LANES,), jnp.float32) for _ in range(NCH)))

    HALF = (NBLK - 1) // 2
    start_gather(1, 1)

    def pair_body(i, carry):
        b0 = 2 * i
        wait_gather(0)
        carry = accumulate(rows[0], b0, carry)
        start_gather(b0 + 2, 0)
        wait_gather(1)
        carry = accumulate(rows[1], b0 + 1, carry)
        start_gather(b0 + 3, 1)
        return carry

    carry = lax.fori_loop(0, HALF - 1, pair_body, carry0)
    b0 = 2 * (HALF - 1)
    wait_gather(0)
    carry = accumulate(rows[0], b0, carry)
    start_gather(b0 + 2, 0)
    wait_gather(1)
    carry = accumulate(rows[1], b0 + 1, carry)
    wait_gather(0)
    carry = accumulate(rows[0], NBLK - 1, carry)
    seg_cur, av = carry
    flush(seg_cur, av)


@jax.jit
def _sc_pool(x_pad, b_pad, table):
    mesh = plsc.VectorSubcoreMesh(core_axis_name="c", subcore_axis_name="s")
    return pl.kernel(
        _sc_pool_body,
        out_type=jax.ShapeDtypeStruct((NC, NS, GP, D), jnp.float32),
        mesh=mesh,
        scratch_types=[
            pltpu.VMEM((NBLK, BLK), jnp.int32),
            pltpu.VMEM((NBLK, BLK), jnp.int32),
            pltpu.VMEM((BLK,), jnp.int32),
            pltpu.VMEM((BLK,), jnp.int32),
            pltpu.VMEM((BLK, D), jnp.float32),
            pltpu.VMEM((BLK, D), jnp.float32),
            pltpu.VMEM((ZR, D), jnp.float32),
            pltpu.VMEM((D,), jnp.float32),
            pltpu.SemaphoreType.DMA,
            pltpu.SemaphoreType.DMA,
            pltpu.SemaphoreType.DMA,
        ],
    )(x_pad, b_pad, table)


def _mlp_body(pp_ref, w1_ref, b1_ref, w2_ref, b2_ref, out_ref):
    pooled = jnp.sum(pp_ref[:, :, :G, :], axis=(0, 1))
    h = jnp.dot(pooled, w1_ref[...], preferred_element_type=jnp.float32)
    h = jnp.maximum(h + b1_ref[...], 0.0)
    logits = jnp.dot(h, w2_ref[...], preferred_element_type=jnp.float32)
    logits = logits + b2_ref[...]
    m = jnp.max(logits, axis=1, keepdims=True)
    e = jnp.exp(logits - m)
    out_ref[...] = e / jnp.sum(e, axis=1, keepdims=True)


@jax.jit
def _mlp(pp, w1, b1, w2, b2):
    return pl.pallas_call(
        _mlp_body,
        out_shape=jax.ShapeDtypeStruct((G, C), jnp.float32),
    )(pp, w1, b1, w2, b2)


def kernel(x, edge_index, batch, table, W1, b1, W2, b2):
    del edge_index
    xf = x.reshape(-1).astype(jnp.int32)
    bf = batch.astype(jnp.int32)
    npad = NP - N
    x_pad = jnp.concatenate([xf, jnp.zeros((npad,), jnp.int32)])
    b_pad = jnp.concatenate([bf, jnp.full((npad,), G, jnp.int32)])
    x_pad = x_pad.reshape(NW, NBLK, BLK)
    b_pad = b_pad.reshape(NW, NBLK, BLK)
    partial = _sc_pool(x_pad, b_pad, table)
    return _mlp(partial, W1, b1.reshape(1, H), W2, b2.reshape(1, C))


# speedup vs baseline: 2.4027x; 2.4027x over previous
import functools

import jax
import jax.numpy as jnp
from jax import lax
from jax.experimental import pallas as pl
from jax.experimental.pallas import tpu as pltpu
from jax.experimental.pallas import tpu_sc as plsc

N = 50000
VOCAB = 100000
D = 256
H = 512
C = 32
G = 512

NC = 2
NS = 16
NW = NC * NS
BLK = 112
NBLK = -(-N // (NW * BLK))
CHUNK = NBLK * BLK
NP = NW * CHUNK
GP = G + 8
LANES = 16
NCH = D // LANES
ZR = 64


def _sc_pool_body(x_hbm, b_hbm, table_hbm, out_hbm,
                  x_v, b_v, xidx0, xidx1, rows0, rows1, zbuf, fbuf,
                  sem0, sem1, zsem):
    c = lax.axis_index("c")
    s = lax.axis_index("s")
    wid = s * NC + c

    pltpu.sync_copy(x_hbm.at[wid], x_v)
    pltpu.sync_copy(b_hbm.at[wid], b_v)

    xidx = (xidx0, xidx1)
    rows = (rows0, rows1)
    sems = (sem0, sem1)

    def copy_idx(row, dst_ref):
        for j in range(BLK // LANES):
            dst_ref[pl.ds(j * LANES, LANES)] = x_v[row, pl.ds(j * LANES, LANES)]

    def start_gather(blk, p):
        copy_idx(blk, xidx[p])
        pltpu.async_copy(table_hbm.at[xidx[p]], rows[p], sems[p])

    def wait_gather(p):
        pltpu.make_async_copy(table_hbm.at[xidx[p]], rows[p], sems[p]).wait()

    zv = jnp.zeros((LANES,), jnp.float32)

    def zero_zbuf_row(r, _):
        for j in range(NCH):
            zbuf[r, pl.ds(j * LANES, LANES)] = zv
        return 0

    lax.fori_loop(0, ZR, zero_zbuf_row, 0)
    start_gather(0, 0)
    for k in range(G // ZR):
        pltpu.async_copy(zbuf, out_hbm.at[c, s, pl.ds(k * ZR, ZR)], zsem)
    for k in range(G // ZR):
        pltpu.make_async_copy(zbuf, out_hbm.at[c, s, pl.ds(k * ZR, ZR)],
                              zsem).wait()

    def flush(seg, av):
        for j in range(NCH):
            fbuf[pl.ds(j * LANES, LANES)] = av[j]
        pltpu.sync_copy(fbuf, out_hbm.at[c, s, seg])

    def accumulate(rows_ref, blk, carry):
        def group(g, carry):
            seg_cur, av = carry
            segs = b_v[blk, pl.ds(g * LANES, LANES)]
            for l in range(LANES):
                seg_l = segs[l]
                is_new = seg_l != seg_cur

                @pl.when(is_new)
                def _(seg_cur=seg_cur, av=av):
                    flush(seg_cur, av)

                keep = 1.0 - is_new.astype(jnp.float32)
                r = g * LANES + l
                av = tuple(av[j] * keep + rows_ref[r, pl.ds(j * LANES, LANES)]
                           for j in range(NCH))
                seg_cur = seg_l
            return (seg_cur, av)

        return lax.fori_loop(0, BLK // LANES, group, carry)

    carry0 = (b_v[0, pl.ds(0, LANES)][0],
              tuple(jnp.zeros((LANES,), jnp.float32) for _ in range(NCH)))

    HALF = NBLK // 2
    start_gather(1, 1)

    def pair_body(i, carry):
        b0 = 2 * i
        wait_gather(0)
        carry = accumulate(rows[0], b0, carry)
        start_gather(b0 + 2, 0)
        wait_gather(1)
        carry = accumulate(rows[1], b0 + 1, carry)
        start_gather(b0 + 3, 1)
        return carry

    carry = lax.fori_loop(0, HALF - 1, pair_body, carry0)
    b0 = 2 * (HALF - 1)
    wait_gather(0)
    carry = accumulate(rows[0], b0, carry)
    wait_gather(1)
    carry = accumulate(rows[1], b0 + 1, carry)
    seg_cur, av = carry
    flush(seg_cur, av)


@jax.jit
def _sc_pool(x_pad, b_pad, table):
    mesh = plsc.VectorSubcoreMesh(core_axis_name="c", subcore_axis_name="s")
    return pl.kernel(
        _sc_pool_body,
        out_type=jax.ShapeDtypeStruct((NC, NS, GP, D), jnp.float32),
        mesh=mesh,
        scratch_types=[
            pltpu.VMEM((NBLK, BLK), jnp.int32),
            pltpu.VMEM((NBLK, BLK), jnp.int32),
            pltpu.VMEM((BLK,), jnp.int32),
            pltpu.VMEM((BLK,), jnp.int32),
            pltpu.VMEM((BLK, D), jnp.float32),
            pltpu.VMEM((BLK, D), jnp.float32),
            pltpu.VMEM((ZR, D), jnp.float32),
            pltpu.VMEM((D,), jnp.float32),
            pltpu.SemaphoreType.DMA,
            pltpu.SemaphoreType.DMA,
            pltpu.SemaphoreType.DMA,
        ],
    )(x_pad, b_pad, table)


def _mlp_body(pp_ref, w1_ref, b1_ref, w2_ref, b2_ref, out_ref):
    pooled = jnp.sum(pp_ref[:, :, :G, :], axis=(0, 1))
    h = jnp.dot(pooled, w1_ref[...], preferred_element_type=jnp.float32)
    h = jnp.maximum(h + b1_ref[...], 0.0)
    logits = jnp.dot(h, w2_ref[...], preferred_element_type=jnp.float32)
    logits = logits + b2_ref[...]
    m = jnp.max(logits, axis=1, keepdims=True)
    e = jnp.exp(logits - m)
    out_ref[...] = e / jnp.sum(e, axis=1, keepdims=True)


@jax.jit
def _mlp(pp, w1, b1, w2, b2):
    return pl.pallas_call(
        _mlp_body,
        out_shape=jax.ShapeDtypeStruct((G, C), jnp.float32),
    )(pp, w1, b1, w2, b2)


def kernel(x, edge_index, batch, table, W1, b1, W2, b2):
    del edge_index
    xf = x.reshape(-1).astype(jnp.int32)
    bf = batch.astype(jnp.int32)
    npad = NP - N
    x_pad = jnp.concatenate([xf, jnp.arange(npad, dtype=jnp.int32)])
    b_pad = jnp.concatenate([bf, jnp.full((npad,), G, jnp.int32)])
    x_pad = x_pad.reshape(NW, NBLK, BLK)
    b_pad = b_pad.reshape(NW, NBLK, BLK)
    partial = _sc_pool(x_pad, b_pad, table)
    return _mlp(partial, W1, b1.reshape(1, H), W2, b2.reshape(1, C))
